# baseline (speedup 1.0000x reference)
import jax
import jax.numpy as jnp
from jax import lax
from jax.experimental import pallas as pl
from jax.experimental.pallas import tpu as pltpu
from jax.experimental.pallas import tpu_sc as plsc

NC = 2
NS = 16
NW = NC * NS
LANES = 16
SUB = 32


def _sc_body(x_hbm, pe_hbm, out_hbm, pe_v, xb0, xb1, ld0, ld1, st0, st1):
    nbatch, nrows, d = x_hbm.shape
    rows_w = nrows // NW
    nsub = rows_w // SUB
    nch = nsub * nbatch

    cid = lax.axis_index("c")
    sid = lax.axis_index("s")
    row0 = (sid * NC + cid) * rows_w

    bufs = (xb0, xb1)
    ldsems = (ld0, ld1)
    stsems = (st0, st1)

    def add_pe(buf):
        def row_body(r, carry):
            for k in range(d // LANES):
                sl = pl.ds(k * LANES, LANES)
                plsc.addupdate(buf.at[r, sl], pe_v[r, sl])
            return carry
        lax.fori_loop(0, SUB, row_body, 0)

    pltpu.sync_copy(pe_hbm.at[pl.ds(row0, SUB)], pe_v)
    loads = [None] * nch
    stores = [None] * nch
    loads[0] = pltpu.async_copy(x_hbm.at[0, pl.ds(row0, SUB)], bufs[0], ldsems[0])
    for ch in range(nch):
        sub, b = divmod(ch, nbatch)
        buf = bufs[ch % 2]
        loads[ch].wait()
        if ch + 1 < nch:
            if ch - 1 >= 0:
                stores[ch - 1].wait()
            nxt_sub, nxt_b = divmod(ch + 1, nbatch)
            loads[ch + 1] = pltpu.async_copy(
                x_hbm.at[nxt_b, pl.ds(row0 + nxt_sub * SUB, SUB)],
                bufs[(ch + 1) % 2], ldsems[(ch + 1) % 2])
        add_pe(buf)
        if b == nbatch - 1 and sub + 1 < nsub:
            pltpu.sync_copy(pe_hbm.at[pl.ds(row0 + (sub + 1) * SUB, SUB)], pe_v)
        stores[ch] = pltpu.async_copy(
            buf, out_hbm.at[b, pl.ds(row0 + sub * SUB, SUB)], stsems[ch % 2])
    stores[nch - 2].wait()
    stores[nch - 1].wait()


def kernel(x, pe_table):
    B, L, D = x.shape
    mesh = plsc.VectorSubcoreMesh(core_axis_name="c", subcore_axis_name="s")
    f = pl.kernel(
        _sc_body,
        mesh=mesh,
        out_type=jax.ShapeDtypeStruct((B, L, D), x.dtype),
        scratch_types=[
            pltpu.VMEM((SUB, D), jnp.float32),
            pltpu.VMEM((SUB, D), jnp.float32),
            pltpu.VMEM((SUB, D), jnp.float32),
            pltpu.SemaphoreType.DMA,
            pltpu.SemaphoreType.DMA,
            pltpu.SemaphoreType.DMA,
            pltpu.SemaphoreType.DMA,
        ],
    )
    return f(x, pe_table)


# speedup vs baseline: 1.7352x; 1.7352x over previous
import jax
import jax.numpy as jnp
from jax import lax
from jax.experimental import pallas as pl
from jax.experimental.pallas import tpu as pltpu
from jax.experimental.pallas import tpu_sc as plsc

NC = 2
NS = 16
NW = NC * NS
LANES = 16
SUB = 32


def _sc_body(x_hbm, pe_hbm, out_hbm, pe_v, xb0, xb1, ld0, ld1, st0, st1):
    nbatch, nrows, d = x_hbm.shape
    rows_w = nrows // NW
    nsub = rows_w // SUB
    nch = nsub * nbatch

    cid = lax.axis_index("c")
    sid = lax.axis_index("s")
    row0 = (sid * NC + cid) * rows_w

    bufs = (xb0, xb1)
    ldsems = (ld0, ld1)
    stsems = (st0, st1)

    def add_pe(buf):
        G = 8

        def row_body(r, carry):
            for g in range(0, d // LANES, G):
                sls = [pl.ds((g + j) * LANES, LANES) for j in range(G)]
                vecs = [pe_v[r, sl] for sl in sls]
                for sl, v in zip(sls, vecs):
                    plsc.addupdate(buf.at[r, sl], v)
            return carry
        lax.fori_loop(0, SUB, row_body, 0)

    pltpu.sync_copy(pe_hbm.at[pl.ds(row0, SUB)], pe_v)
    loads = [None] * nch
    stores = [None] * nch
    loads[0] = pltpu.async_copy(x_hbm.at[0, pl.ds(row0, SUB)], bufs[0], ldsems[0])
    for ch in range(nch):
        sub, b = divmod(ch, nbatch)
        buf = bufs[ch % 2]
        loads[ch].wait()
        if ch + 1 < nch:
            if ch - 1 >= 0:
                stores[ch - 1].wait()
            nxt_sub, nxt_b = divmod(ch + 1, nbatch)
            loads[ch + 1] = pltpu.async_copy(
                x_hbm.at[nxt_b, pl.ds(row0 + nxt_sub * SUB, SUB)],
                bufs[(ch + 1) % 2], ldsems[(ch + 1) % 2])
        add_pe(buf)
        if b == nbatch - 1 and sub + 1 < nsub:
            pltpu.sync_copy(pe_hbm.at[pl.ds(row0 + (sub + 1) * SUB, SUB)], pe_v)
        stores[ch] = pltpu.async_copy(
            buf, out_hbm.at[b, pl.ds(row0 + sub * SUB, SUB)], stsems[ch % 2])
    stores[nch - 2].wait()
    stores[nch - 1].wait()


def kernel(x, pe_table):
    B, L, D = x.shape
    mesh = plsc.VectorSubcoreMesh(core_axis_name="c", subcore_axis_name="s")
    f = pl.kernel(
        _sc_body,
        mesh=mesh,
        out_type=jax.ShapeDtypeStruct((B, L, D), x.dtype),
        scratch_types=[
            pltpu.VMEM((SUB, D), jnp.float32),
            pltpu.VMEM((SUB, D), jnp.float32),
            pltpu.VMEM((SUB, D), jnp.float32),
            pltpu.SemaphoreType.DMA,
            pltpu.SemaphoreType.DMA,
            pltpu.SemaphoreType.DMA,
            pltpu.SemaphoreType.DMA,
        ],
    )
    return f(x, pe_table)
